# baseline (speedup 1.0000x reference)
import functools

import jax
import jax.numpy as jnp
from jax import lax
from jax.experimental import pallas as pl
from jax.experimental.pallas import tpu as pltpu
from jax.experimental.pallas import tpu_sc as plsc

R = 128
N = 32768
NC = 2
NS = 16
L = 16
NW = NC * NS
K = 8

SC_ROWS = 0


def _make_sc_argmin(nrows):
    rows_per_w = nrows // NW
    outer = N // (K * L)

    @functools.partial(
        pl.kernel,
        mesh=plsc.VectorSubcoreMesh(core_axis_name="c", subcore_axis_name="s"),
        out_type=jax.ShapeDtypeStruct((NW * 8,), jnp.int32),
        scratch_types=[
            pltpu.VMEM((2, N), jnp.float32),
            pltpu.VMEM((16,), jnp.int32),
            pltpu.SemaphoreType.DMA,
            pltpu.SemaphoreType.DMA,
        ],
    )
    def sc_argmin(x_hbm, out_hbm, rows_v, res_v, sem0, sem1):
        wid = lax.axis_index("s") * NC + lax.axis_index("c")
        row0 = wid * rows_per_w

        lane = lax.iota(jnp.int32, L)
        res = jnp.zeros((L,), jnp.int32)
        sems = (sem0, sem1)

        def merge(av, ai, bv, bi):
            take = (bv < av) | ((bv == av) & (bi < ai))
            return jnp.where(take, bv, av), jnp.where(take, bi, ai)

        cps = [None, None]
        cps[0] = pltpu.async_copy(x_hbm.at[row0], rows_v.at[0], sems[0])

        for r in range(rows_per_w):
            b = r % 2
            cps[b].wait()
            if r + 1 < rows_per_w:
                nb = (r + 1) % 2
                cps[nb] = pltpu.async_copy(
                    x_hbm.at[row0 + r + 1], rows_v.at[nb], sems[nb]
                )

            buf = rows_v.at[b]

            def body(t, carry):
                bvs, bts = map(list, carry)
                base = t * (K * L)
                tv = jnp.full((L,), t, jnp.int32)
                for j in range(K):
                    v = buf[pl.ds(base + j * L, L)]
                    m = v < bvs[j]
                    bvs[j] = jnp.minimum(v, bvs[j])
                    bts[j] = jnp.where(m, tv, bts[j])
                return tuple(bvs), tuple(bts)

            init = (
                tuple(jnp.full((L,), jnp.inf, jnp.float32) for _ in range(K)),
                tuple(jnp.zeros((L,), jnp.int32) for _ in range(K)),
            )
            bvs, bts = lax.fori_loop(0, outer, body, init)

            bvs, bis = list(bvs), [
                bts[j] * (K * L) + (j * L) + lane for j in range(K)
            ]
            step = 1
            while step < K:
                for j in range(0, K, 2 * step):
                    bvs[j], bis[j] = merge(
                        bvs[j], bis[j], bvs[j + step], bis[j + step]
                    )
                step *= 2
            bestv, besti = bvs[0], bis[0]

            bm = bestv[0]
            bi = besti[0]
            for j in range(1, L):
                v = bestv[j]
                i = besti[j]
                take = (v < bm) | ((v == bm) & (i < bi))
                bm = jnp.where(take, v, bm)
                bi = jnp.where(take, i, bi)
            res = jnp.where(lane == r, bi, res)

        res_v[...] = res
        pltpu.sync_copy(res_v.at[pl.ds(0, 8)], out_hbm.at[pl.ds(wid * 8, 8)])

    return sc_argmin


RBLK = 64
CCHUNK = 1024


def _tc_body(xl_ref, xr_ref, o_ref):
    col0 = lax.broadcasted_iota(jnp.int32, (RBLK, CCHUNK), 1)
    half_c = (N // 2) // CCHUNK

    def make_body(ref, coff):
        def body(c, carry):
            bestv, bestt = carry
            v = ref[:, pl.ds(c * CCHUNK, CCHUNK)]
            m = v < bestv
            bestv = jnp.minimum(v, bestv)
            bestt = jnp.where(
                m, jnp.full((RBLK, CCHUNK), c + coff, jnp.int32), bestt
            )
            return bestv, bestt
        return body

    init = (
        jnp.full((RBLK, CCHUNK), jnp.inf, jnp.float32),
        jnp.zeros((RBLK, CCHUNK), jnp.int32),
    )
    carry = lax.fori_loop(0, half_c, make_body(xl_ref, 0), init, unroll=8)
    bestv, bestt = lax.fori_loop(
        0, half_c, make_body(xr_ref, half_c), carry, unroll=8
    )

    besti = bestt * CCHUNK + col0
    mv = jnp.min(bestv, axis=1, keepdims=True)
    cand = jnp.where(bestv == mv, besti, jnp.int32(2**31 - 1))
    o_ref[...] = jnp.min(cand, axis=1).reshape(1, 1, RBLK)


def _make_tc_argmin(row_start, nrows):
    grid = nrows // RBLK
    blk0 = row_start // RBLK
    return pl.pallas_call(
        _tc_body,
        grid=(grid,),
        in_specs=[
            pl.BlockSpec((RBLK, N // 2), lambda i: (i + blk0, 0)),
            pl.BlockSpec((RBLK, N // 2), lambda i: (i + blk0, 1)),
        ],
        out_specs=pl.BlockSpec((1, 1, RBLK), lambda i: (i, 0, 0)),
        out_shape=jax.ShapeDtypeStruct((grid, 1, RBLK), jnp.int32),
    )


def kernel(x):
    parts = []
    if SC_ROWS > 0:
        sc_out = _make_sc_argmin(SC_ROWS)(x)
        rpw = SC_ROWS // NW
        parts.append(sc_out.reshape(NW, 8)[:, :rpw].reshape(SC_ROWS))
    if SC_ROWS < R:
        tc_out = _make_tc_argmin(SC_ROWS, R - SC_ROWS)(x, x)
        parts.append(tc_out.reshape(R - SC_ROWS))
    return parts[0] if len(parts) == 1 else jnp.concatenate(parts)


# speedup vs baseline: 1.0763x; 1.0763x over previous
import functools

import jax
import jax.numpy as jnp
from jax import lax
from jax.experimental import pallas as pl
from jax.experimental.pallas import tpu as pltpu
from jax.experimental.pallas import tpu_sc as plsc

R = 128
N = 32768
NC = 2
NS = 16
L = 16
NW = NC * NS
K = 8

SC_ROWS = 0


def _make_sc_argmin(nrows):
    rows_per_w = nrows // NW
    outer = N // (K * L)

    @functools.partial(
        pl.kernel,
        mesh=plsc.VectorSubcoreMesh(core_axis_name="c", subcore_axis_name="s"),
        out_type=jax.ShapeDtypeStruct((NW * 8,), jnp.int32),
        scratch_types=[
            pltpu.VMEM((2, N), jnp.float32),
            pltpu.VMEM((16,), jnp.int32),
            pltpu.SemaphoreType.DMA,
            pltpu.SemaphoreType.DMA,
        ],
    )
    def sc_argmin(x_hbm, out_hbm, rows_v, res_v, sem0, sem1):
        wid = lax.axis_index("s") * NC + lax.axis_index("c")
        row0 = wid * rows_per_w

        lane = lax.iota(jnp.int32, L)
        res = jnp.zeros((L,), jnp.int32)
        sems = (sem0, sem1)

        def merge(av, ai, bv, bi):
            take = (bv < av) | ((bv == av) & (bi < ai))
            return jnp.where(take, bv, av), jnp.where(take, bi, ai)

        cps = [None, None]
        cps[0] = pltpu.async_copy(x_hbm.at[row0], rows_v.at[0], sems[0])

        for r in range(rows_per_w):
            b = r % 2
            cps[b].wait()
            if r + 1 < rows_per_w:
                nb = (r + 1) % 2
                cps[nb] = pltpu.async_copy(
                    x_hbm.at[row0 + r + 1], rows_v.at[nb], sems[nb]
                )

            buf = rows_v.at[b]

            def body(t, carry):
                bvs, bts = map(list, carry)
                base = t * (K * L)
                tv = jnp.full((L,), t, jnp.int32)
                for j in range(K):
                    v = buf[pl.ds(base + j * L, L)]
                    m = v < bvs[j]
                    bvs[j] = jnp.minimum(v, bvs[j])
                    bts[j] = jnp.where(m, tv, bts[j])
                return tuple(bvs), tuple(bts)

            init = (
                tuple(jnp.full((L,), jnp.inf, jnp.float32) for _ in range(K)),
                tuple(jnp.zeros((L,), jnp.int32) for _ in range(K)),
            )
            bvs, bts = lax.fori_loop(0, outer, body, init)

            bvs, bis = list(bvs), [
                bts[j] * (K * L) + (j * L) + lane for j in range(K)
            ]
            step = 1
            while step < K:
                for j in range(0, K, 2 * step):
                    bvs[j], bis[j] = merge(
                        bvs[j], bis[j], bvs[j + step], bis[j + step]
                    )
                step *= 2
            bestv, besti = bvs[0], bis[0]

            bm = bestv[0]
            bi = besti[0]
            for j in range(1, L):
                v = bestv[j]
                i = besti[j]
                take = (v < bm) | ((v == bm) & (i < bi))
                bm = jnp.where(take, v, bm)
                bi = jnp.where(take, i, bi)
            res = jnp.where(lane == r, bi, res)

        res_v[...] = res
        pltpu.sync_copy(res_v.at[pl.ds(0, 8)], out_hbm.at[pl.ds(wid * 8, 8)])

    return sc_argmin


RBLK = 64
CCHUNK = 512


def _tc_body(x0_ref, x1_ref, x2_ref, x3_ref, o_ref):
    col0 = lax.broadcasted_iota(jnp.int32, (RBLK, CCHUNK), 1)
    quart_c = (N // 4) // CCHUNK

    def make_body(ref, coff):
        def body(c, carry):
            bestv, bestt = carry
            v = ref[:, pl.ds(c * CCHUNK, CCHUNK)]
            m = v < bestv
            bestv = jnp.minimum(v, bestv)
            bestt = jnp.where(
                m, jnp.full((RBLK, CCHUNK), c + coff, jnp.int32), bestt
            )
            return bestv, bestt
        return body

    carry = (
        jnp.full((RBLK, CCHUNK), jnp.inf, jnp.float32),
        jnp.zeros((RBLK, CCHUNK), jnp.int32),
    )
    for q, ref in enumerate((x0_ref, x1_ref, x2_ref, x3_ref)):
        carry = lax.fori_loop(
            0, quart_c, make_body(ref, q * quart_c), carry, unroll=8
        )
    bestv, bestt = carry

    besti = bestt * CCHUNK + col0
    mv = jnp.min(bestv, axis=1, keepdims=True)
    cand = jnp.where(bestv == mv, besti, jnp.int32(2**31 - 1))
    o_ref[...] = jnp.min(cand, axis=1).reshape(1, 1, RBLK)


def _make_tc_argmin(row_start, nrows):
    grid = nrows // RBLK
    blk0 = row_start // RBLK
    return pl.pallas_call(
        _tc_body,
        grid=(grid,),
        in_specs=[
            pl.BlockSpec((RBLK, N // 4), lambda i, q=q: (i + blk0, q))
            for q in range(4)
        ],
        out_specs=pl.BlockSpec((1, 1, RBLK), lambda i: (i, 0, 0)),
        out_shape=jax.ShapeDtypeStruct((grid, 1, RBLK), jnp.int32),
    )


def kernel(x):
    parts = []
    if SC_ROWS > 0:
        sc_out = _make_sc_argmin(SC_ROWS)(x)
        rpw = SC_ROWS // NW
        parts.append(sc_out.reshape(NW, 8)[:, :rpw].reshape(SC_ROWS))
    if SC_ROWS < R:
        tc_out = _make_tc_argmin(SC_ROWS, R - SC_ROWS)(x, x, x, x)
        parts.append(tc_out.reshape(R - SC_ROWS))
    return parts[0] if len(parts) == 1 else jnp.concatenate(parts)
